# Initial kernel scaffold; baseline (speedup 1.0000x reference)
import jax
import jax.numpy as jnp
from jax.experimental import pallas as pl


def kernel(hidden_states, gate_w, w1, b1, w2, b2):
    raise NotImplementedError("write your pallas kernel here")


# speedup vs baseline: 2.4907x; 2.4907x over previous
import functools

import jax
import jax.numpy as jnp
from jax.experimental import pallas as pl
from jax.experimental.pallas import tpu as pltpu

_HIDDEN = 1024
_N_EXPERTS = 8
_D_FF = 4096

_M_BLK = 1024
_FF_BLK = 512
_N_FF = _D_FF // _FF_BLK


def _dense_moe_kernel(x_ref, gw_ref, w1_ref, b1_ref, w2_ref, b2_ref,
                      out_ref, wts_ref, acc_ref):
    e = pl.program_id(1)
    ff = pl.program_id(2)

    @pl.when((e == 0) & (ff == 0))
    def _gate():
        logits = jnp.dot(x_ref[...], gw_ref[...].T,
                         preferred_element_type=jnp.float32)
        iota = jax.lax.broadcasted_iota(jnp.int32, logits.shape, 1)
        a1 = jnp.argmax(logits, axis=1)
        l1 = jnp.max(logits, axis=1, keepdims=True)
        masked = jnp.where(iota == a1[:, None], -jnp.inf, logits)
        a2 = jnp.argmax(masked, axis=1)
        l2 = jnp.max(masked, axis=1, keepdims=True)
        p1 = jax.nn.sigmoid(l1 - l2)
        wts_ref[...] = (jnp.where(iota == a1[:, None], p1, 0.0)
                        + jnp.where(iota == a2[:, None], 1.0 - p1, 0.0))

    @pl.when(ff == 0)
    def _zero():
        acc_ref[...] = jnp.zeros_like(acc_ref)

    h = jnp.dot(x_ref[...], w1_ref[0].T, preferred_element_type=jnp.float32)
    h = h + b1_ref[0]
    h = 0.5 * h * (1.0 + jax.lax.erf(h * 0.7071067811865476))
    acc_ref[...] += jnp.dot(h, w2_ref[0].T, preferred_element_type=jnp.float32)

    @pl.when(ff == _N_FF - 1)
    def _combine():
        eiota = jax.lax.broadcasted_iota(jnp.int32, wts_ref.shape, 1)
        wcol = jnp.sum(jnp.where(eiota == e, wts_ref[...], 0.0), axis=1,
                       keepdims=True)
        contrib = (acc_ref[...] + b2_ref[0]) * wcol

        @pl.when(e == 0)
        def _():
            out_ref[...] = contrib

        @pl.when(e != 0)
        def _():
            out_ref[...] += contrib


def kernel(hidden_states, gate_w, w1, b1, w2, b2):
    B, S, H = hidden_states.shape
    flat = hidden_states.reshape(-1, H)
    T = flat.shape[0]

    grid = (T // _M_BLK, _N_EXPERTS, _N_FF)
    out = pl.pallas_call(
        _dense_moe_kernel,
        grid=grid,
        in_specs=[
            pl.BlockSpec((_M_BLK, H), lambda m, e, ff: (m, 0)),
            pl.BlockSpec((_N_EXPERTS, H), lambda m, e, ff: (0, 0)),
            pl.BlockSpec((1, _FF_BLK, H), lambda m, e, ff: (e, ff, 0)),
            pl.BlockSpec((1, 1, _FF_BLK), lambda m, e, ff: (e, 0, ff)),
            pl.BlockSpec((1, H, _FF_BLK), lambda m, e, ff: (e, 0, ff)),
            pl.BlockSpec((1, 1, H), lambda m, e, ff: (e, 0, 0)),
        ],
        out_specs=pl.BlockSpec((_M_BLK, H), lambda m, e, ff: (m, 0)),
        out_shape=jax.ShapeDtypeStruct((T, H), jnp.float32),
        scratch_shapes=[
            pltpu.VMEM((_M_BLK, _N_EXPERTS), jnp.float32),
            pltpu.VMEM((_M_BLK, H), jnp.float32),
        ],
        compiler_params=pltpu.CompilerParams(
            dimension_semantics=("parallel", "arbitrary", "arbitrary"),
        ),
    )(flat, gate_w, w1, b1.reshape(_N_EXPERTS, 1, _D_FF),
      w2, b2.reshape(_N_EXPERTS, 1, H))
    return out.reshape(B, S, H)
